# baseline (speedup 1.0000x reference)
import functools

import jax
import jax.numpy as jnp
from jax import lax
from jax.experimental import pallas as pl
from jax.experimental.pallas import tpu as pltpu
from jax.experimental.pallas import tpu_sc as plsc

CH = 8


def _sc_add_kernel(B, S, H, NC, NS):
    NW = NC * NS
    rows_per_w = S // NW
    n_chunks = rows_per_w // CH
    mesh = plsc.VectorSubcoreMesh(core_axis_name="c", subcore_axis_name="s")

    @functools.partial(
        pl.kernel,
        mesh=mesh,
        out_type=jax.ShapeDtypeStruct((B, S, H), jnp.float32),
        scratch_types=(
            [pltpu.VMEM((CH, H), jnp.float32) for _ in range(2)]
            + [pltpu.VMEM((CH, H), jnp.float32) for _ in range(2 * B)]
            + [pltpu.SemaphoreType.DMA for _ in range(2)]
            + [pltpu.SemaphoreType.DMA for _ in range(2 * B)]
            + [pltpu.SemaphoreType.DMA for _ in range(2 * B)]
        ),
    )
    def k(x_hbm, pe_hbm, out_hbm, *scr):
        pbs = scr[0:2]
        xbs = scr[2 : 2 + 2 * B]
        sps = scr[2 + 2 * B : 4 + 2 * B]
        sis = scr[4 + 2 * B : 4 + 4 * B]
        sos = scr[4 + 4 * B : 4 + 6 * B]
        wid = lax.axis_index("s") * NC + lax.axis_index("c")
        seq0 = wid * rows_per_w

        def x_src(b, base):
            return x_hbm.at[b, pl.ds(base, CH)]

        def pe_src(base):
            return pe_hbm.at[pl.ds(base, CH)]

        pltpu.async_copy(pe_src(seq0), pbs[0], sps[0])
        pltpu.async_copy(pe_src(seq0 + CH), pbs[1], sps[1])
        for b in range(B):
            pltpu.async_copy(x_src(b, seq0), xbs[b], sis[b])

        def half_body(cc, carry):
            for i in range(2):
                c = 2 * cc + i
                base = seq0 + c * CH
                pe_v = pbs[i]
                pltpu.make_async_copy(pe_src(base), pe_v, sps[i]).wait()
                for b in range(B):
                    p = i * B + b
                    pn = (1 - i) * B + b
                    xb = xbs[p]
                    pltpu.make_async_copy(x_src(b, base), xb, sis[p]).wait()

                    @pl.when(c > 0)
                    def _(b=b, pn=pn, base=base):
                        pltpu.make_async_copy(
                            xbs[pn], out_hbm.at[b, pl.ds(base - CH, CH)], sos[pn]
                        ).wait()
                        nbase = jnp.minimum(base + CH, seq0 + (n_chunks - 1) * CH)
                        pltpu.async_copy(x_src(b, nbase), xbs[pn], sis[pn])

                    @pl.when(c == 0)
                    def _(b=b, pn=pn, base=base):
                        pltpu.async_copy(x_src(b, base + CH), xbs[pn], sis[pn])

                    def row_body(r, carry2, xb=xb, pe_v=pe_v):
                        @plsc.parallel_loop(0, H // 16, unroll=8)
                        def jloop(j):
                            sl = pl.ds(j * 16, 16)
                            plsc.addupdate(xb.at[r, sl], pe_v[r, sl])

                        return carry2

                    lax.fori_loop(0, CH, row_body, 0)

                    pltpu.async_copy(xb, out_hbm.at[b, pl.ds(base, CH)], sos[p])

                nbase = seq0 + jnp.minimum(c + 2, n_chunks - 1) * CH
                pltpu.async_copy(pe_src(nbase), pe_v, sps[i])
            return carry

        lax.fori_loop(0, n_chunks // 2, half_body, 0)

        last = seq0 + (n_chunks - 1) * CH
        for b in range(B):
            pltpu.make_async_copy(
                xbs[B + b], out_hbm.at[b, pl.ds(last, CH)], sos[B + b]
            ).wait()
            pltpu.make_async_copy(x_src(b, last), xbs[b], sis[b]).wait()
        pltpu.make_async_copy(pe_src(last), pbs[0], sps[0]).wait()
        pltpu.make_async_copy(pe_src(last), pbs[1], sps[1]).wait()

    return k


def kernel(x, pe):
    B, S, H = x.shape
    info = plsc.get_sparse_core_info()
    k = _sc_add_kernel(B, S, H, info.num_cores, info.num_subcores)
    return k(x, pe)
